# Initial kernel scaffold; baseline (speedup 1.0000x reference)
import jax
import jax.numpy as jnp
from jax.experimental import pallas as pl


def kernel(x_nchw, conv1_w, conv1_b, conv2_w, conv2_b, conv3_w, conv3_b, head_w, head_b, sel1, sel2, sel3):
    raise NotImplementedError("write your pallas kernel here")


# speedup vs baseline: 3.4340x; 3.4340x over previous
from functools import partial

import jax
import jax.numpy as jnp
from jax.experimental import pallas as pl
from jax.experimental.pallas import tpu as pltpu

_VMEM = 60 * 1024 * 1024
_DT = jnp.float32


def _conv1_pool_kernel(x_ref, w_ref, b_ref, o_ref):
    bt = x_ref.shape[1]
    lo, hi = [], []
    for g in range(5):
        xs = x_ref[g * 96:(g + 1) * 96].reshape(96 * bt, 54)
        y = jnp.dot(xs, w_ref[...], preferred_element_type=jnp.float32)
        y = jnp.maximum(y + b_ref[...], 0.0).reshape(3, 32, bt, 256)
        y = jnp.maximum(jnp.maximum(y[0], y[1]), y[2])
        y = y[:30].reshape(10, 3, bt, 256)
        y = jnp.maximum(jnp.maximum(y[:, 0], y[:, 1]), y[:, 2])
        lo.append(y[..., :128])
        hi.append(y[..., 128:])
    o_ref[...] = jnp.concatenate(lo + hi, axis=0).astype(o_ref.dtype)


def _stage2_kernel(x_ref, w2_ref, b2_ref, w3_ref, b3_ref, wh_ref, bh_ref,
                   o_ref):
    bt = x_ref.shape[1]
    acc2 = jnp.zeros((78 * bt, 256), jnp.float32)
    for t in range(9):
        off = (t // 3) * 10 + t % 3
        xs = x_ref[off:off + 78].reshape(78 * bt, 128)
        acc2 = acc2 + jnp.dot(xs, w2_ref[t],
                              preferred_element_type=jnp.float32)
    y2 = jnp.maximum(acc2 + b2_ref[...], 0.0).astype(x_ref.dtype)
    y2 = y2.reshape(78, bt, 256)

    acc3 = jnp.zeros((56 * bt, 384), jnp.float32)
    for t in range(9):
        off = (t // 3) * 10 + t % 3
        xs = y2[off:off + 56].reshape(56 * bt, 256)
        acc3 = acc3 + jnp.dot(xs, w3_ref[t],
                              preferred_element_type=jnp.float32)
    y3 = jnp.maximum(acc3 + b3_ref[...], 0.0).astype(x_ref.dtype)
    y3 = y3.reshape(56, bt, 384)

    feats = jnp.concatenate(
        [y3[oh * 10 + ow] for oh in range(6) for ow in range(6)], axis=1)
    out = jnp.dot(feats, wh_ref[...], preferred_element_type=jnp.float32)
    o_ref[...] = out + bh_ref[...]


def kernel(x_nchw, conv1_w, conv1_b, conv2_w, conv2_b, conv3_w, conv3_b,
           head_w, head_b, sel1, sel2, sel3):
    B = x_nchw.shape[0]
    bt1 = 64 if B % 64 == 0 else B
    bt2 = 64 if B % 64 == 0 else B

    xsp = jnp.transpose(x_nchw.astype(jnp.float32), (2, 3, 0, 1))
    xsp = xsp.reshape(32 * 32, B, 3)
    xsp = jnp.pad(xsp, ((0, 2), (0, 0), (0, 0))).astype(_DT)
    offs = [kh * 32 + kw for kh in range(3) for kw in range(3)]
    x1 = jnp.concatenate(
        [xsp[o:o + 480] for o in offs]
        + [xsp[480 + o:960 + o] for o in offs], axis=2)

    w1 = jnp.zeros((54, 256), jnp.float32)
    w1 = w1.at[:27, :128].set(conv1_w).at[27:, 128:].set(conv1_w).astype(_DT)
    b1 = jnp.concatenate([conv1_b, conv1_b], axis=1)

    pooled = pl.pallas_call(
        _conv1_pool_kernel,
        out_shape=jax.ShapeDtypeStruct((100, B, 128), _DT),
        grid=(B // bt1,),
        in_specs=[
            pl.BlockSpec((480, bt1, 54), lambda i: (0, i, 0)),
            pl.BlockSpec((54, 256), lambda i: (0, 0)),
            pl.BlockSpec((1, 256), lambda i: (0, 0)),
        ],
        out_specs=pl.BlockSpec((100, bt1, 128), lambda i: (0, i, 0)),
        compiler_params=pltpu.CompilerParams(
            dimension_semantics=("parallel",), vmem_limit_bytes=_VMEM),
    )(x1, w1, b1)

    w2r = conv2_w.reshape(9, 128, 256).astype(_DT)
    w3r = jnp.pad(conv3_w.reshape(9, 256, 300),
                  ((0, 0), (0, 0), (0, 84))).astype(_DT)
    b3p = jnp.pad(conv3_b, ((0, 0), (0, 84)))
    whr = jnp.pad(head_w.reshape(36, 300, 10),
                  ((0, 0), (0, 84), (0, 0))).reshape(36 * 384, 10).astype(_DT)

    out = pl.pallas_call(
        _stage2_kernel,
        out_shape=jax.ShapeDtypeStruct((B, 10), jnp.float32),
        grid=(B // bt2,),
        in_specs=[
            pl.BlockSpec((100, bt2, 128), lambda i: (0, i, 0)),
            pl.BlockSpec((9, 128, 256), lambda i: (0, 0, 0)),
            pl.BlockSpec((1, 256), lambda i: (0, 0)),
            pl.BlockSpec((9, 256, 384), lambda i: (0, 0, 0)),
            pl.BlockSpec((1, 384), lambda i: (0, 0)),
            pl.BlockSpec((36 * 384, 10), lambda i: (0, 0)),
            pl.BlockSpec((1, 10), lambda i: (0, 0)),
        ],
        out_specs=pl.BlockSpec((bt2, 10), lambda i: (i, 0)),
        compiler_params=pltpu.CompilerParams(
            dimension_semantics=("parallel",), vmem_limit_bytes=_VMEM),
    )(pooled, w2r, conv2_b, w3r, b3p, whr, head_b)
    return out


# speedup vs baseline: 4.1228x; 1.2006x over previous
from functools import partial

import jax
import jax.numpy as jnp
from jax.experimental import pallas as pl
from jax.experimental.pallas import tpu as pltpu

_VMEM = 60 * 1024 * 1024
_DT = jnp.bfloat16


def _conv1_pool_kernel(x_ref, w_ref, b_ref, o_ref):
    bt = x_ref.shape[1]
    lo, hi = [], []
    for g in range(5):
        xs = x_ref[g * 96:(g + 1) * 96].reshape(96 * bt, 54)
        y = jnp.dot(xs, w_ref[...], preferred_element_type=jnp.float32)
        y = jnp.maximum(y + b_ref[...], 0.0).reshape(3, 32, bt, 256)
        y = jnp.maximum(jnp.maximum(y[0], y[1]), y[2])
        y = y[:30].reshape(10, 3, bt, 256)
        y = jnp.maximum(jnp.maximum(y[:, 0], y[:, 1]), y[:, 2])
        lo.append(y[..., :128])
        hi.append(y[..., 128:])
    o_ref[...] = jnp.concatenate(lo + hi, axis=0).astype(o_ref.dtype)


def _stage2_kernel(x_ref, w2_ref, b2_ref, w3_ref, b3_ref, wh_ref, bh_ref,
                   o_ref):
    bt = x_ref.shape[1]
    acc2 = jnp.zeros((78 * bt, 256), jnp.float32)
    for t in range(9):
        off = (t // 3) * 10 + t % 3
        xs = x_ref[off:off + 78].reshape(78 * bt, 128)
        acc2 = acc2 + jnp.dot(xs, w2_ref[t],
                              preferred_element_type=jnp.float32)
    y2 = jnp.maximum(acc2 + b2_ref[...], 0.0).astype(x_ref.dtype)
    y2 = y2.reshape(78, bt, 256)

    acc3 = jnp.zeros((56 * bt, 384), jnp.float32)
    for t in range(9):
        off = (t // 3) * 10 + t % 3
        xs = y2[off:off + 56].reshape(56 * bt, 256)
        acc3 = acc3 + jnp.dot(xs, w3_ref[t],
                              preferred_element_type=jnp.float32)
    y3 = jnp.maximum(acc3 + b3_ref[...], 0.0).astype(x_ref.dtype)
    y3 = y3.reshape(56, bt, 384)

    feats = jnp.concatenate(
        [y3[oh * 10 + ow] for oh in range(6) for ow in range(6)], axis=1)
    out = jnp.dot(feats, wh_ref[...], preferred_element_type=jnp.float32)
    o_ref[...] = out + bh_ref[...]


def kernel(x_nchw, conv1_w, conv1_b, conv2_w, conv2_b, conv3_w, conv3_b,
           head_w, head_b, sel1, sel2, sel3):
    B = x_nchw.shape[0]
    bt1 = 64 if B % 64 == 0 else B
    bt2 = 64 if B % 64 == 0 else B

    xsp = jnp.transpose(x_nchw.astype(jnp.float32), (2, 3, 0, 1))
    xsp = xsp.reshape(32 * 32, B, 3)
    xsp = jnp.pad(xsp, ((0, 2), (0, 0), (0, 0))).astype(_DT)
    offs = [kh * 32 + kw for kh in range(3) for kw in range(3)]
    x1 = jnp.concatenate(
        [xsp[o:o + 480] for o in offs]
        + [xsp[480 + o:960 + o] for o in offs], axis=2)

    w1 = jnp.zeros((54, 256), jnp.float32)
    w1 = w1.at[:27, :128].set(conv1_w).at[27:, 128:].set(conv1_w).astype(_DT)
    b1 = jnp.concatenate([conv1_b, conv1_b], axis=1)

    pooled = pl.pallas_call(
        _conv1_pool_kernel,
        out_shape=jax.ShapeDtypeStruct((100, B, 128), _DT),
        grid=(B // bt1,),
        in_specs=[
            pl.BlockSpec((480, bt1, 54), lambda i: (0, i, 0)),
            pl.BlockSpec((54, 256), lambda i: (0, 0)),
            pl.BlockSpec((1, 256), lambda i: (0, 0)),
        ],
        out_specs=pl.BlockSpec((100, bt1, 128), lambda i: (0, i, 0)),
        compiler_params=pltpu.CompilerParams(
            dimension_semantics=("parallel",), vmem_limit_bytes=_VMEM),
    )(x1, w1, b1)

    w2r = conv2_w.reshape(9, 128, 256).astype(_DT)
    w3r = jnp.pad(conv3_w.reshape(9, 256, 300),
                  ((0, 0), (0, 0), (0, 84))).astype(_DT)
    b3p = jnp.pad(conv3_b, ((0, 0), (0, 84)))
    whr = jnp.pad(head_w.reshape(36, 300, 10),
                  ((0, 0), (0, 84), (0, 0))).reshape(36 * 384, 10).astype(_DT)

    out = pl.pallas_call(
        _stage2_kernel,
        out_shape=jax.ShapeDtypeStruct((B, 10), jnp.float32),
        grid=(B // bt2,),
        in_specs=[
            pl.BlockSpec((100, bt2, 128), lambda i: (0, i, 0)),
            pl.BlockSpec((9, 128, 256), lambda i: (0, 0, 0)),
            pl.BlockSpec((1, 256), lambda i: (0, 0)),
            pl.BlockSpec((9, 256, 384), lambda i: (0, 0, 0)),
            pl.BlockSpec((1, 384), lambda i: (0, 0)),
            pl.BlockSpec((36 * 384, 10), lambda i: (0, 0)),
            pl.BlockSpec((1, 10), lambda i: (0, 0)),
        ],
        out_specs=pl.BlockSpec((bt2, 10), lambda i: (i, 0)),
        compiler_params=pltpu.CompilerParams(
            dimension_semantics=("parallel",), vmem_limit_bytes=_VMEM),
    )(pooled, w2r, conv2_b, w3r, b3p, whr, head_b)
    return out


# speedup vs baseline: 5.9201x; 1.4359x over previous
from functools import partial

import jax
import jax.numpy as jnp
from jax.experimental import pallas as pl
from jax.experimental.pallas import tpu as pltpu

_VMEM = 60 * 1024 * 1024
_DT = jnp.bfloat16


def _conv1_pool_kernel(x_ref, w_ref, b_ref, o_ref):
    bt = x_ref.shape[1]
    lo, hi = [], []
    for g in range(5):
        xs = x_ref[g * 96:(g + 1) * 96].reshape(96 * bt, 54)
        y = jnp.dot(xs, w_ref[...], preferred_element_type=jnp.float32)
        y = jnp.maximum(y + b_ref[...], 0.0).reshape(3, 32, bt, 256)
        y = jnp.maximum(jnp.maximum(y[0], y[1]), y[2])
        y = y[:30].reshape(10, 3, bt, 256)
        y = jnp.maximum(jnp.maximum(y[:, 0], y[:, 1]), y[:, 2])
        lo.append(y[..., :128])
        hi.append(y[..., 128:])
    o_ref[...] = jnp.concatenate(lo + hi, axis=0).astype(o_ref.dtype)


def _stage2_kernel(x_ref, w2_ref, b2_ref, w3_ref, b3_ref, wh_ref, bh_ref,
                   o_ref):
    bt = x_ref.shape[1]
    offs = [(t // 3) * 10 + t % 3 for t in range(9)]
    xs2 = jnp.concatenate(
        [x_ref[o:o + 78].reshape(78 * bt, 128) for o in offs], axis=1)
    acc2 = jnp.dot(xs2, w2_ref[...], preferred_element_type=jnp.float32)
    y2 = jnp.maximum(acc2 + b2_ref[...], 0.0).astype(x_ref.dtype)
    y2 = y2.reshape(78, bt, 256)

    xs3 = jnp.concatenate(
        [y2[o:o + 56].reshape(56 * bt, 256) for o in offs], axis=1)
    acc3 = jnp.dot(xs3, w3_ref[...], preferred_element_type=jnp.float32)
    y3 = jnp.maximum(acc3 + b3_ref[...], 0.0).astype(x_ref.dtype)
    y3 = y3.reshape(56, bt, 384)

    feats = jnp.concatenate(
        [y3[oh * 10 + ow] for oh in range(6) for ow in range(6)], axis=1)
    out = jnp.dot(feats, wh_ref[...], preferred_element_type=jnp.float32)
    o_ref[...] = out + bh_ref[...]


def kernel(x_nchw, conv1_w, conv1_b, conv2_w, conv2_b, conv3_w, conv3_b,
           head_w, head_b, sel1, sel2, sel3):
    B = x_nchw.shape[0]
    bt1 = 64 if B % 64 == 0 else B
    bt2 = 64 if B % 64 == 0 else B

    xsp = jnp.transpose(x_nchw.astype(jnp.float32), (2, 3, 0, 1))
    xsp = xsp.reshape(32 * 32, B, 3)
    xsp = jnp.pad(xsp, ((0, 2), (0, 0), (0, 0))).astype(_DT)
    offs = [kh * 32 + kw for kh in range(3) for kw in range(3)]
    x1 = jnp.concatenate(
        [xsp[o:o + 480] for o in offs]
        + [xsp[480 + o:960 + o] for o in offs], axis=2)

    w1 = jnp.zeros((54, 256), jnp.float32)
    w1 = w1.at[:27, :128].set(conv1_w).at[27:, 128:].set(conv1_w).astype(_DT)
    b1 = jnp.concatenate([conv1_b, conv1_b], axis=1)

    pooled = pl.pallas_call(
        _conv1_pool_kernel,
        out_shape=jax.ShapeDtypeStruct((100, B, 128), _DT),
        grid=(B // bt1,),
        in_specs=[
            pl.BlockSpec((480, bt1, 54), lambda i: (0, i, 0)),
            pl.BlockSpec((54, 256), lambda i: (0, 0)),
            pl.BlockSpec((1, 256), lambda i: (0, 0)),
        ],
        out_specs=pl.BlockSpec((100, bt1, 128), lambda i: (0, i, 0)),
        compiler_params=pltpu.CompilerParams(
            dimension_semantics=("parallel",), vmem_limit_bytes=_VMEM),
    )(x1, w1, b1)

    w2r = conv2_w.astype(_DT)
    w3r = jnp.pad(conv3_w, ((0, 0), (0, 84))).astype(_DT)
    b3p = jnp.pad(conv3_b, ((0, 0), (0, 84)))
    whr = jnp.pad(head_w.reshape(36, 300, 10),
                  ((0, 0), (0, 84), (0, 0))).reshape(36 * 384, 10).astype(_DT)

    out = pl.pallas_call(
        _stage2_kernel,
        out_shape=jax.ShapeDtypeStruct((B, 10), jnp.float32),
        grid=(B // bt2,),
        in_specs=[
            pl.BlockSpec((100, bt2, 128), lambda i: (0, i, 0)),
            pl.BlockSpec((1152, 256), lambda i: (0, 0)),
            pl.BlockSpec((1, 256), lambda i: (0, 0)),
            pl.BlockSpec((2304, 384), lambda i: (0, 0)),
            pl.BlockSpec((1, 384), lambda i: (0, 0)),
            pl.BlockSpec((36 * 384, 10), lambda i: (0, 0)),
            pl.BlockSpec((1, 10), lambda i: (0, 0)),
        ],
        out_specs=pl.BlockSpec((bt2, 10), lambda i: (i, 0)),
        compiler_params=pltpu.CompilerParams(
            dimension_semantics=("parallel",), vmem_limit_bytes=_VMEM),
    )(pooled, w2r, conv2_b, w3r, b3p, whr, head_b)
    return out
